# baseline (speedup 1.0000x reference)
import functools

import jax
import jax.numpy as jnp
from jax.experimental import pallas as pl
from jax.experimental.pallas import tpu as pltpu
from jax.experimental.pallas import tpu_sc as plsc

TM = 2000
TMA = 200
GW = 120


def _elu(x):
    return jnp.where(x > 0, x, jnp.exp(jnp.minimum(x, 0.0)) - 1.0)


def _agg1_body(adj_ref, f_ref, w1_ref, b_ref, w2_ref, o_ref, a8_ref, x1_ref):
    @pl.when(pl.program_id(0) == 0)
    def _():
        x1_ref[...] = jnp.dot(f_ref[...].astype(jnp.bfloat16), w1_ref[...],
                              preferred_element_type=jnp.float32
                              ).astype(jnp.bfloat16)

    a = adj_ref[...]
    a8_ref[...] = a.astype(jnp.float8_e4m3fn)
    acc = jnp.dot(a.astype(jnp.bfloat16), x1_ref[...],
                  preferred_element_type=jnp.float32)
    h = _elu(acc + b_ref[...])
    o_ref[...] = jnp.dot(h.astype(jnp.bfloat16), w2_ref[...],
                         preferred_element_type=jnp.float32
                         ).astype(jnp.bfloat16)


def _agg2_body(adj_ref, x_ref, b_ref, o_ref):
    acc = jnp.dot(adj_ref[...].astype(jnp.bfloat16), x_ref[...],
                  preferred_element_type=jnp.float32)
    h = _elu(_elu(acc + b_ref[...]))
    nrm = jnp.sqrt(jnp.sum(h * h, axis=1, keepdims=True))
    hn = h / jnp.maximum(nrm, 1e-8)
    d2 = hn.shape[1] // 2
    lo = jax.lax.bitcast_convert_type(
        hn[:, :d2].astype(jnp.bfloat16).astype(jnp.float32), jnp.int32)
    hi = jax.lax.bitcast_convert_type(
        hn[:, d2:].astype(jnp.bfloat16).astype(jnp.float32), jnp.int32)
    o_ref[...] = jnp.bitwise_or(
        jax.lax.shift_right_logical(lo, 16),
        jnp.bitwise_and(hi, jnp.int32(-65536)))


def _unpack_pair(w32):
    lo = jax.lax.bitcast_convert_type(
        jax.lax.shift_left(w32, 16), jnp.float32)
    hi = jax.lax.bitcast_convert_type(
        jnp.bitwise_and(w32, jnp.int32(-65536)), jnp.float32)
    return lo, hi


def _loss_body(hn_ref, g_ref, o_ref, n_rows):
    i = pl.program_id(0)
    hn_lo, hn_hi = _unpack_pair(hn_ref[...])
    g_lo, g_hi = _unpack_pair(g_ref[...])
    sims = (jnp.sum(hn_lo[None, :, :] * g_lo, axis=2)
            + jnp.sum(hn_hi[None, :, :] * g_hi, axis=2))
    lse = jnp.log(jnp.sum(jnp.exp(sims[1:, :]), axis=0))

    @pl.when(i == 0)
    def _():
        o_ref[0, 0] = 0.0

    o_ref[0, 0] += jnp.sum(lse - sims[0, :]) / n_rows


def _sc_gather(hn, idx_flat, n_gather, d):
    mesh = plsc.VectorSubcoreMesh(core_axis_name="core",
                                  subcore_axis_name="subcore")

    n_blocks = n_gather // GW

    @pl.kernel(out_type=jax.ShapeDtypeStruct((n_gather, d), hn.dtype),
               mesh=mesh)
    def kern(x_hbm, i_hbm, o_hbm):
        def body(i_vmem, o_vmem):
            pltpu.sync_copy(x_hbm.at[i_vmem.at[0, 0]], o_vmem)

        pltpu.emit_pipeline(
            body,
            grid=(n_blocks,),
            in_specs=[pl.BlockSpec((1, 1, GW), index_map=lambda i: (i, 0, 0))],
            out_specs=[pl.BlockSpec((GW, d), index_map=lambda i: (i, 0))],
            core_axis_name=("core", "subcore"),
            dimension_semantics=(pltpu.PARALLEL,),
        )(i_hbm, o_hbm)

    return kern(hn, idx_flat.reshape(n_blocks, 1, GW))


def kernel(feature, adj, sample, W1, b1, W2, b2):
    n, d_in = feature.shape
    d_h = W1.shape[1]
    w = sample.shape[1]
    nm = n // TM
    nma = n // TMA

    w1b = W1.astype(jnp.bfloat16)
    w2b = W2.astype(jnp.bfloat16)
    b1r = b1.reshape(1, d_h)
    b2r = b2.reshape(1, d_h)

    x2 = pl.pallas_call(
        _agg1_body,
        grid=(nma,),
        in_specs=[
            pl.BlockSpec((TMA, n), lambda m: (m, 0)),
            pl.BlockSpec((n, d_in), lambda m: (0, 0)),
            pl.BlockSpec((d_in, d_h), lambda m: (0, 0)),
            pl.BlockSpec((1, d_h), lambda m: (0, 0)),
            pl.BlockSpec((d_h, d_h), lambda m: (0, 0)),
        ],
        out_specs=[
            pl.BlockSpec((TMA, d_h), lambda m: (m, 0)),
            pl.BlockSpec((TMA, n), lambda m: (m, 0)),
        ],
        out_shape=[
            jax.ShapeDtypeStruct((n, d_h), jnp.bfloat16),
            jax.ShapeDtypeStruct((n, n), jnp.float8_e4m3fn),
        ],
        scratch_shapes=[pltpu.VMEM((n, d_h), jnp.bfloat16)],
    )(adj, feature, w1b, b1r, w2b)
    x2, adj8 = x2

    d2 = d_h // 2
    hn32 = pl.pallas_call(
        _agg2_body,
        grid=(nma,),
        in_specs=[
            pl.BlockSpec((TMA, n), lambda m: (m, 0)),
            pl.BlockSpec((n, d_h), lambda m: (0, 0)),
            pl.BlockSpec((1, d_h), lambda m: (0, 0)),
        ],
        out_specs=pl.BlockSpec((TMA, d2), lambda m: (m, 0)),
        out_shape=jax.ShapeDtypeStruct((n, d2), jnp.int32),
    )(adj8, x2, b2r)

    idx_flat = sample.astype(jnp.int32).T.reshape(1, n * w)
    g32 = _sc_gather(hn32, idx_flat, n * w, d2)
    g32 = g32.reshape(w, n, d2)

    tt = 1000
    loss = pl.pallas_call(
        functools.partial(_loss_body, n_rows=float(n)),
        grid=(n // tt,),
        in_specs=[
            pl.BlockSpec((tt, d2), lambda i: (i, 0)),
            pl.BlockSpec((w, tt, d2), lambda i: (0, i, 0)),
        ],
        out_specs=pl.BlockSpec(memory_space=pltpu.SMEM),
        out_shape=jax.ShapeDtypeStruct((1, 1), jnp.float32),
    )(hn32, g32)

    return loss.reshape(())


# speedup vs baseline: 1.0296x; 1.0296x over previous
import functools

import jax
import jax.numpy as jnp
from jax.experimental import pallas as pl
from jax.experimental.pallas import tpu as pltpu
from jax.experimental.pallas import tpu_sc as plsc

TM = 2000
TMA = 200
GW = 240


def _elu(x):
    return jnp.where(x > 0, x, jnp.exp(jnp.minimum(x, 0.0)) - 1.0)


def _agg1_body(adj_ref, f_ref, w1_ref, b_ref, w2_ref, o_ref, a8_ref, x1_ref):
    @pl.when(pl.program_id(0) == 0)
    def _():
        x1_ref[...] = jnp.dot(f_ref[...].astype(jnp.bfloat16), w1_ref[...],
                              preferred_element_type=jnp.float32
                              ).astype(jnp.bfloat16)

    a = adj_ref[...]
    a8_ref[...] = a.astype(jnp.float8_e4m3fn)
    acc = jnp.dot(a.astype(jnp.bfloat16), x1_ref[...],
                  preferred_element_type=jnp.float32)
    h = _elu(acc + b_ref[...])
    x2 = jnp.dot(h.astype(jnp.bfloat16), w2_ref[...],
                 preferred_element_type=jnp.float32)
    o_ref[...] = (x2 * 0.0625).astype(jnp.float8_e4m3fn)


def _agg2_body(adj_ref, x_ref, b_ref, o_ref):
    acc = jnp.dot(adj_ref[...], x_ref[...],
                  preferred_element_type=jnp.float32) * 16.0
    h = _elu(_elu(acc + b_ref[...]))
    nrm = jnp.sqrt(jnp.sum(h * h, axis=1, keepdims=True))
    hn = h / jnp.maximum(nrm, 1e-8)
    d2 = hn.shape[1] // 2
    lo = jax.lax.bitcast_convert_type(
        hn[:, :d2].astype(jnp.bfloat16).astype(jnp.float32), jnp.int32)
    hi = jax.lax.bitcast_convert_type(
        hn[:, d2:].astype(jnp.bfloat16).astype(jnp.float32), jnp.int32)
    o_ref[...] = jnp.bitwise_or(
        jax.lax.shift_right_logical(lo, 16),
        jnp.bitwise_and(hi, jnp.int32(-65536)))


def _unpack_pair(w32):
    lo = jax.lax.bitcast_convert_type(
        jax.lax.shift_left(w32, 16), jnp.float32)
    hi = jax.lax.bitcast_convert_type(
        jnp.bitwise_and(w32, jnp.int32(-65536)), jnp.float32)
    return lo, hi


def _loss_body(hn_ref, g_ref, o_ref, n_rows):
    i = pl.program_id(0)
    hn_lo, hn_hi = _unpack_pair(hn_ref[...])
    g_lo, g_hi = _unpack_pair(g_ref[...])
    sims = (jnp.sum(hn_lo[None, :, :] * g_lo, axis=2)
            + jnp.sum(hn_hi[None, :, :] * g_hi, axis=2))
    lse = jnp.log(jnp.sum(jnp.exp(sims[1:, :]), axis=0))

    @pl.when(i == 0)
    def _():
        o_ref[0, 0] = 0.0

    o_ref[0, 0] += jnp.sum(lse - sims[0, :]) / n_rows


def _sc_gather(hn, idx_flat, n_gather, d):
    mesh = plsc.VectorSubcoreMesh(core_axis_name="core",
                                  subcore_axis_name="subcore")

    n_blocks = n_gather // GW

    @pl.kernel(out_type=jax.ShapeDtypeStruct((n_gather, d), hn.dtype),
               mesh=mesh)
    def kern(x_hbm, i_hbm, o_hbm):
        def body(i_vmem, o_vmem):
            pltpu.sync_copy(x_hbm.at[i_vmem.at[0, 0]], o_vmem)

        pltpu.emit_pipeline(
            body,
            grid=(n_blocks,),
            in_specs=[pl.BlockSpec((1, 1, GW), index_map=lambda i: (i, 0, 0))],
            out_specs=[pl.BlockSpec((GW, d), index_map=lambda i: (i, 0))],
            core_axis_name=("core", "subcore"),
            dimension_semantics=(pltpu.PARALLEL,),
        )(i_hbm, o_hbm)

    return kern(hn, idx_flat.reshape(n_blocks, 1, GW))


def kernel(feature, adj, sample, W1, b1, W2, b2):
    n, d_in = feature.shape
    d_h = W1.shape[1]
    w = sample.shape[1]
    nm = n // TM
    nma = n // TMA

    w1b = W1.astype(jnp.bfloat16)
    w2b = W2.astype(jnp.bfloat16)
    b1r = b1.reshape(1, d_h)
    b2r = b2.reshape(1, d_h)

    x2 = pl.pallas_call(
        _agg1_body,
        grid=(nma,),
        in_specs=[
            pl.BlockSpec((TMA, n), lambda m: (m, 0)),
            pl.BlockSpec((n, d_in), lambda m: (0, 0)),
            pl.BlockSpec((d_in, d_h), lambda m: (0, 0)),
            pl.BlockSpec((1, d_h), lambda m: (0, 0)),
            pl.BlockSpec((d_h, d_h), lambda m: (0, 0)),
        ],
        out_specs=[
            pl.BlockSpec((TMA, d_h), lambda m: (m, 0)),
            pl.BlockSpec((TMA, n), lambda m: (m, 0)),
        ],
        out_shape=[
            jax.ShapeDtypeStruct((n, d_h), jnp.float8_e4m3fn),
            jax.ShapeDtypeStruct((n, n), jnp.float8_e4m3fn),
        ],
        scratch_shapes=[pltpu.VMEM((n, d_h), jnp.bfloat16)],
    )(adj, feature, w1b, b1r, w2b)
    x2, adj8 = x2

    d2 = d_h // 2
    hn32 = pl.pallas_call(
        _agg2_body,
        grid=(nma,),
        in_specs=[
            pl.BlockSpec((TMA, n), lambda m: (m, 0)),
            pl.BlockSpec((n, d_h), lambda m: (0, 0)),
            pl.BlockSpec((1, d_h), lambda m: (0, 0)),
        ],
        out_specs=pl.BlockSpec((TMA, d2), lambda m: (m, 0)),
        out_shape=jax.ShapeDtypeStruct((n, d2), jnp.int32),
    )(adj8, x2, b2r)

    idx_flat = sample.astype(jnp.int32).T.reshape(1, n * w)
    g32 = _sc_gather(hn32, idx_flat, n * w, d2)
    g32 = g32.reshape(w, n, d2)

    tt = 1000
    loss = pl.pallas_call(
        functools.partial(_loss_body, n_rows=float(n)),
        grid=(n // tt,),
        in_specs=[
            pl.BlockSpec((tt, d2), lambda i: (i, 0)),
            pl.BlockSpec((w, tt, d2), lambda i: (0, i, 0)),
        ],
        out_specs=pl.BlockSpec(memory_space=pltpu.SMEM),
        out_shape=jax.ShapeDtypeStruct((1, 1), jnp.float32),
    )(hn32, g32)

    return loss.reshape(())
